# baseline (speedup 1.0000x reference)
import functools

import jax
import jax.numpy as jnp
from jax import lax
from jax.experimental import pallas as pl
from jax.experimental.pallas import tpu as pltpu
from jax.experimental.pallas import tpu_sc as plsc

B = 4096
L = 200
DIM = 32

NC = 2
NS = 16
NW = NC * NS

N_TOTAL = 2 * B * L
PER_W = N_TOTAL // NW
CH = 128
NCH = PER_W // CH


@functools.partial(
    pl.kernel,
    mesh=plsc.VectorSubcoreMesh(core_axis_name="c", subcore_axis_name="s"),
    out_type=jax.ShapeDtypeStruct((N_TOTAL, DIM), jnp.float32),
    scratch_types=[
        pltpu.VMEM((NCH, CH), jnp.int32),
        pltpu.VMEM((2, CH, DIM), jnp.float32),
        pltpu.SemaphoreType.DMA,
        pltpu.SemaphoreType.DMA,
    ],
    compiler_params=pltpu.CompilerParams(use_tc_tiling_on_sc=False),
)
def _gather_kernel(ids_hbm, table_hbm, out_hbm, idx_v, rows_v, gsem, osem):
    wid = lax.axis_index("s") * NC + lax.axis_index("c")
    base = wid * PER_W
    pltpu.sync_copy(ids_hbm.at[wid], idx_v)

    def body(g, _):
        for b in range(2):
            c = 2 * g + b
            pltpu.async_copy(
                table_hbm.at[idx_v.at[c]], rows_v.at[b], gsem
            ).wait()
            pltpu.async_copy(
                rows_v.at[b], out_hbm.at[pl.ds(base + c * CH, CH)], osem
            ).wait()
        return 0

    lax.fori_loop(0, NCH // 2, body, 0, unroll=False)


def kernel(input_ids, mlm_input_ids, attention_mask, table):
    ids = jnp.concatenate(
        [input_ids.reshape(-1), mlm_input_ids.reshape(-1)]
    ).astype(jnp.int32)
    ids3 = ids.reshape(NW, NCH, CH)
    flat = _gather_kernel(ids3, table)
    emb = flat.reshape(2, B, L, DIM)
    return (emb[0], emb[1], attention_mask)


# speedup vs baseline: 1.1216x; 1.1216x over previous
import functools

import jax
import jax.numpy as jnp
from jax import lax
from jax.experimental import pallas as pl
from jax.experimental.pallas import tpu as pltpu
from jax.experimental.pallas import tpu_sc as plsc

B = 4096
L = 200
DIM = 32

NC = 2
NS = 16
NW = NC * NS

N_TOTAL = 2 * B * L
PER_W = N_TOTAL // NW
CH = 128
NCH = PER_W // CH
K = 4
NG = NCH // K
NT = NG // 2


@functools.partial(
    pl.kernel,
    mesh=plsc.VectorSubcoreMesh(core_axis_name="c", subcore_axis_name="s"),
    out_type=jax.ShapeDtypeStruct((N_TOTAL, DIM), jnp.float32),
    scratch_types=[
        pltpu.VMEM((NCH, CH), jnp.int32),
        pltpu.VMEM((2, K, CH, DIM), jnp.float32),
        pltpu.SemaphoreType.DMA,
        pltpu.SemaphoreType.DMA,
    ],
    compiler_params=pltpu.CompilerParams(use_tc_tiling_on_sc=False),
)
def _gather_kernel(ids_hbm, table_hbm, out_hbm, idx_v, rows_v, gsem, osem):
    wid = lax.axis_index("s") * NC + lax.axis_index("c")
    base = wid * PER_W
    pltpu.sync_copy(ids_hbm.at[wid], idx_v)

    def fire_gather(c, p, j):
        pltpu.async_copy(table_hbm.at[idx_v.at[c]], rows_v.at[p].at[j], gsem)

    def wait_gather(p, j):
        pltpu.make_async_copy(
            table_hbm.at[idx_v.at[0]], rows_v.at[p].at[j], gsem
        ).wait()

    def fire_wb(c, p, j):
        pltpu.async_copy(
            rows_v.at[p].at[j], out_hbm.at[pl.ds(base + c * CH, CH)], osem
        )

    def wait_wb(p, j):
        pltpu.make_async_copy(
            rows_v.at[p].at[j], out_hbm.at[pl.ds(base, CH)], osem
        ).wait()

    for j in range(K):
        fire_gather(j, 0, j)

    def body(t, _):
        for parity in range(2):
            g = 2 * t + parity
            slot, other = parity, 1 - parity

            def _free_other(other=other):
                for j in range(K):
                    wait_wb(other, j)

            def _fire_next(g=g, other=other):
                for j in range(K):
                    fire_gather((g + 1) * K + j, other, j)

            if parity == 0:
                pl.when(t > 0)(_free_other)
                _fire_next()
            else:
                _free_other()
                pl.when(g + 1 < NG)(_fire_next)
            for j in range(K):
                wait_gather(slot, j)
                fire_wb(g * K + j, slot, j)
        return 0

    lax.fori_loop(0, NT, body, 0, unroll=False)
    for j in range(K):
        wait_wb(1, j)


def kernel(input_ids, mlm_input_ids, attention_mask, table):
    ids = jnp.concatenate(
        [input_ids.reshape(-1), mlm_input_ids.reshape(-1)]
    ).astype(jnp.int32)
    ids3 = ids.reshape(NW, NCH, CH)
    flat = _gather_kernel(ids3, table)
    emb = flat.reshape(2, B, L, DIM)
    return (emb[0], emb[1], attention_mask)


# speedup vs baseline: 1.9847x; 1.7696x over previous
import functools

import jax
import jax.numpy as jnp
from jax import lax
from jax.experimental import pallas as pl
from jax.experimental.pallas import tpu as pltpu
from jax.experimental.pallas import tpu_sc as plsc

B = 4096
L = 200
DIM = 32

NC = 2
NS = 16
NW = NC * NS

N_ROWS = B * L
PER_W = N_ROWS // NW
CH = 128
NCH = PER_W // CH
K = 4
NG = NCH // K
NT = NG // 2


@functools.partial(
    pl.kernel,
    mesh=plsc.VectorSubcoreMesh(core_axis_name="c", subcore_axis_name="s"),
    out_type=(
        jax.ShapeDtypeStruct((N_ROWS, DIM), jnp.float32),
        jax.ShapeDtypeStruct((N_ROWS, DIM), jnp.float32),
    ),
    scratch_types=[
        pltpu.VMEM((NCH, CH), jnp.int32),
        pltpu.VMEM((2, K, CH, DIM), jnp.float32),
        pltpu.SemaphoreType.DMA,
        pltpu.SemaphoreType.DMA,
    ],
    compiler_params=pltpu.CompilerParams(use_tc_tiling_on_sc=False),
)
def _gather_kernel(ids0_hbm, ids1_hbm, table_hbm, out0_hbm, out1_hbm,
                   idx_v, rows_v, gsem, osem):
    wid = lax.axis_index("s") * NC + lax.axis_index("c")
    base = wid * PER_W

    def run_phase(ids_hbm, out_hbm):
        pltpu.sync_copy(ids_hbm.at[wid], idx_v)

        def fire_gather(c, p, j):
            pltpu.async_copy(table_hbm.at[idx_v.at[c]], rows_v.at[p].at[j],
                             gsem)

        def wait_gather(p, j):
            pltpu.make_async_copy(
                table_hbm.at[idx_v.at[0]], rows_v.at[p].at[j], gsem
            ).wait()

        def fire_wb(c, p, j):
            pltpu.async_copy(
                rows_v.at[p].at[j], out_hbm.at[pl.ds(base + c * CH, CH)], osem
            )

        def wait_wb(p, j):
            pltpu.make_async_copy(
                rows_v.at[p].at[j], out_hbm.at[pl.ds(base, CH)], osem
            ).wait()

        for j in range(K):
            fire_gather(j, 0, j)

        def body(t, _):
            for parity in range(2):
                g = 2 * t + parity
                slot, other = parity, 1 - parity

                def _free_other(other=other):
                    for j in range(K):
                        wait_wb(other, j)

                def _fire_next(g=g, other=other):
                    for j in range(K):
                        fire_gather((g + 1) * K + j, other, j)

                if parity == 0:
                    pl.when(t > 0)(_free_other)
                    _fire_next()
                else:
                    _free_other()
                    pl.when(g + 1 < NG)(_fire_next)
                for j in range(K):
                    wait_gather(slot, j)
                    fire_wb(g * K + j, slot, j)
            return 0

        lax.fori_loop(0, NT, body, 0, unroll=False)
        for j in range(K):
            wait_wb(1, j)

    run_phase(ids0_hbm, out0_hbm)
    run_phase(ids1_hbm, out1_hbm)


def kernel(input_ids, mlm_input_ids, attention_mask, table):
    ids0 = input_ids.astype(jnp.int32).reshape(NW, NCH, CH)
    ids1 = mlm_input_ids.astype(jnp.int32).reshape(NW, NCH, CH)
    out0, out1 = _gather_kernel(ids0, ids1, table)
    return (out0.reshape(B, L, DIM), out1.reshape(B, L, DIM), attention_mask)


# speedup vs baseline: 1.9892x; 1.0023x over previous
import functools

import jax
import jax.numpy as jnp
from jax import lax
from jax.experimental import pallas as pl
from jax.experimental.pallas import tpu as pltpu
from jax.experimental.pallas import tpu_sc as plsc

B = 4096
L = 200
DIM = 32

NC = 2
NS = 16
NW = NC * NS

ROWS_W = B // NW
K = 4
NG = ROWS_W // K
NT = NG // 2


@functools.partial(
    pl.kernel,
    mesh=plsc.VectorSubcoreMesh(core_axis_name="c", subcore_axis_name="s"),
    out_type=(
        jax.ShapeDtypeStruct((B, L, DIM), jnp.float32),
        jax.ShapeDtypeStruct((B, L, DIM), jnp.float32),
    ),
    scratch_types=[
        pltpu.VMEM((ROWS_W, L), jnp.int32),
        pltpu.VMEM((2, K, L, DIM), jnp.float32),
        pltpu.SemaphoreType.DMA,
        pltpu.SemaphoreType.DMA,
    ],
    compiler_params=pltpu.CompilerParams(use_tc_tiling_on_sc=False),
)
def _gather_kernel(ids0_hbm, ids1_hbm, table_hbm, out0_hbm, out1_hbm,
                   idx_v, rows_v, gsem, osem):
    wid = lax.axis_index("s") * NC + lax.axis_index("c")
    base = wid * ROWS_W

    def run_phase(ids_hbm, out_hbm):
        pltpu.sync_copy(ids_hbm.at[pl.ds(base, ROWS_W)], idx_v)

        def fire_gather(c, p, j):
            pltpu.async_copy(table_hbm.at[idx_v.at[c]], rows_v.at[p].at[j],
                             gsem)

        def wait_gather(p, j):
            pltpu.make_async_copy(
                table_hbm.at[idx_v.at[0]], rows_v.at[p].at[j], gsem
            ).wait()

        def fire_wb(c, p, j):
            pltpu.async_copy(rows_v.at[p].at[j], out_hbm.at[base + c], osem)

        def wait_wb(p, j):
            pltpu.make_async_copy(
                rows_v.at[p].at[j], out_hbm.at[base], osem
            ).wait()

        for j in range(K):
            fire_gather(j, 0, j)

        def body(t, _):
            for parity in range(2):
                g = 2 * t + parity
                slot, other = parity, 1 - parity

                def _free_other(other=other):
                    for j in range(K):
                        wait_wb(other, j)

                def _fire_next(g=g, other=other):
                    for j in range(K):
                        fire_gather((g + 1) * K + j, other, j)

                if parity == 0:
                    pl.when(t > 0)(_free_other)
                    _fire_next()
                else:
                    _free_other()
                    pl.when(g + 1 < NG)(_fire_next)
                for j in range(K):
                    wait_gather(slot, j)
                    fire_wb(g * K + j, slot, j)
            return 0

        lax.fori_loop(0, NT, body, 0, unroll=False)
        for j in range(K):
            wait_wb(1, j)

    run_phase(ids0_hbm, out0_hbm)
    run_phase(ids1_hbm, out1_hbm)


def kernel(input_ids, mlm_input_ids, attention_mask, table):
    out0, out1 = _gather_kernel(input_ids.astype(jnp.int32),
                                mlm_input_ids.astype(jnp.int32), table)
    return (out0, out1, attention_mask)
